# baseline (speedup 1.0000x reference)
import jax
import jax.numpy as jnp
from jax import lax
from jax.experimental import pallas as pl
from jax.experimental.pallas import tpu as pltpu
from jax.experimental.pallas import tpu_sc as plsc


def _sc_body(table_hbm, med_hbm, mole_hbm, out_hbm, mole_v, idx_v, rows_v, sem):
    nc = 2
    wid = lax.axis_index("s") * nc + lax.axis_index("c")
    b_per_w = mole_v.shape[0]
    base = wid * b_per_w
    pltpu.sync_copy(mole_hbm.at[pl.ds(base, b_per_w)], mole_v)
    pltpu.async_copy(med_hbm.at[mole_v], idx_v, sem).wait()
    pltpu.async_copy(table_hbm.at[idx_v], rows_v, sem).wait()
    pltpu.sync_copy(rows_v, out_hbm.at[pl.ds(base, b_per_w)])


def kernel(relation_matrix, med, mole):
    b = mole.shape[0]
    d = relation_matrix.shape[1]
    n_workers = 32
    b_per_w = b // n_workers
    mesh = plsc.VectorSubcoreMesh(core_axis_name="c", subcore_axis_name="s")
    k = pl.kernel(
        _sc_body,
        mesh=mesh,
        compiler_params=pltpu.CompilerParams(use_tc_tiling_on_sc=False),
        out_type=jax.ShapeDtypeStruct((b, d), relation_matrix.dtype),
        scratch_types=[
            pltpu.VMEM((b_per_w,), jnp.int32),
            pltpu.VMEM((b_per_w,), jnp.int32),
            pltpu.VMEM((b_per_w, d), jnp.float32),
            pltpu.SemaphoreType.DMA,
        ],
    )
    return k(relation_matrix, med, mole)


# speedup vs baseline: 2.4400x; 2.4400x over previous
import jax
import jax.numpy as jnp
from jax import lax
from jax.experimental import pallas as pl
from jax.experimental.pallas import tpu as pltpu
from jax.experimental.pallas import tpu_sc as plsc


def _sc_body(table_hbm, med_hbm, mole_hbm, out_hbm, med_v, mole_v, idx_v, sem):
    nc = 2
    wid = lax.axis_index("s") * nc + lax.axis_index("c")
    b_per_w = 128
    base = wid * b_per_w
    pltpu.sync_copy(med_hbm, med_v)
    pltpu.sync_copy(mole_hbm.at[pl.ds(base, b_per_w)], mole_v)
    for t in range(b_per_w // 16):
        m16 = mole_v[pl.ds(t * 16, 16)]
        i16 = plsc.load_gather(med_v, [m16])
        idx_v[pl.ds(t * 16, 16)] = i16
    for t in range(b_per_w // 16):
        vblk = idx_v[pl.ds(t * 16, 16)]
        copies = []
        for jj in range(16):
            g = vblk[jj]
            copies.append(
                pltpu.async_copy(table_hbm.at[g],
                                 out_hbm.at[base + t * 16 + jj], sem))
        for cp in copies:
            cp.wait()


def kernel(relation_matrix, med, mole):
    b = mole.shape[0]
    d = relation_matrix.shape[1]
    mesh = plsc.VectorSubcoreMesh(core_axis_name="c", subcore_axis_name="s")
    k = pl.kernel(
        _sc_body,
        mesh=mesh,
        compiler_params=pltpu.CompilerParams(needs_layout_passes=False),
        out_type=jax.ShapeDtypeStruct((b, d), relation_matrix.dtype),
        scratch_types=[
            pltpu.VMEM((b,), jnp.int32),
            pltpu.VMEM((128,), jnp.int32),
            pltpu.VMEM((128,), jnp.int32),
            pltpu.SemaphoreType.DMA,
        ],
    )
    return k(relation_matrix, med, mole)


# speedup vs baseline: 2.4411x; 1.0005x over previous
import jax
import jax.numpy as jnp
from jax import lax
from jax.experimental import pallas as pl
from jax.experimental.pallas import tpu as pltpu
from jax.experimental.pallas import tpu_sc as plsc


def _sc_body(table_hbm, med_hbm, mole_hbm, out_hbm, med_v, mole_v, idx_v, sem):
    nc = 2
    wid = lax.axis_index("s") * nc + lax.axis_index("c")
    b_per_w = 128
    base = wid * b_per_w
    pltpu.sync_copy(med_hbm, med_v)
    pltpu.sync_copy(mole_hbm.at[pl.ds(base, b_per_w)], mole_v)
    for t in range(b_per_w // 16):
        m16 = mole_v[pl.ds(t * 16, 16)]
        i16 = plsc.load_gather(med_v, [m16])
        idx_v[pl.ds(t * 16, 16)] = i16
    copies = []
    for t in range(b_per_w // 16):
        vblk = idx_v[pl.ds(t * 16, 16)]
        for jj in range(16):
            g = vblk[jj]
            copies.append(
                pltpu.async_copy(table_hbm.at[g],
                                 out_hbm.at[base + t * 16 + jj], sem))
    for cp in copies:
        cp.wait()


def kernel(relation_matrix, med, mole):
    b = mole.shape[0]
    d = relation_matrix.shape[1]
    mesh = plsc.VectorSubcoreMesh(core_axis_name="c", subcore_axis_name="s")
    k = pl.kernel(
        _sc_body,
        mesh=mesh,
        compiler_params=pltpu.CompilerParams(needs_layout_passes=False),
        out_type=jax.ShapeDtypeStruct((b, d), relation_matrix.dtype),
        scratch_types=[
            pltpu.VMEM((b,), jnp.int32),
            pltpu.VMEM((128,), jnp.int32),
            pltpu.VMEM((128,), jnp.int32),
            pltpu.SemaphoreType.DMA,
        ],
    )
    return k(relation_matrix, med, mole)


# speedup vs baseline: 5.4487x; 2.2321x over previous
import jax
import jax.numpy as jnp
from jax import lax
from jax.experimental import pallas as pl
from jax.experimental.pallas import tpu as pltpu
from jax.experimental.pallas import tpu_sc as plsc

_CHUNK = 32
_BPW = 128


def _sc_body(table_hbm, med_hbm, mole_hbm, out_hbm, med_v, mole_v, idx_v,
             buf0, buf1, sem_in0, sem_in1, sem_out0, sem_out1):
    nc = 2
    wid = lax.axis_index("s") * nc + lax.axis_index("c")
    base = wid * _BPW
    pltpu.sync_copy(med_hbm, med_v)
    pltpu.sync_copy(mole_hbm.at[pl.ds(base, _BPW)], mole_v)
    for t in range(_BPW // 16):
        m16 = mole_v[pl.ds(t * 16, 16)]
        idx_v[pl.ds(t * 16, 16)] = plsc.load_gather(med_v, [m16])

    bufs = (buf0, buf1)
    sems_in = (sem_in0, sem_in1)
    sems_out = (sem_out0, sem_out1)
    n_chunks = _BPW // _CHUNK
    out_copies = [None, None]
    for c in range(n_chunks):
        slot = c % 2
        if out_copies[slot] is not None:
            out_copies[slot].wait()
        gathers = []
        for t in range(_CHUNK // 16):
            vblk = idx_v[pl.ds(c * _CHUNK + t * 16, 16)]
            for jj in range(16):
                g = vblk[jj]
                gathers.append(
                    pltpu.make_async_copy(table_hbm.at[g],
                                          bufs[slot].at[t * 16 + jj],
                                          sems_in[slot]))
        for cp in gathers:
            cp.start()
        for cp in gathers:
            cp.wait()
        oc = pltpu.make_async_copy(
            bufs[slot], out_hbm.at[pl.ds(base + c * _CHUNK, _CHUNK)],
            sems_out[slot])
        oc.start()
        out_copies[slot] = oc
    for oc in out_copies:
        if oc is not None:
            oc.wait()


def kernel(relation_matrix, med, mole):
    b = mole.shape[0]
    d = relation_matrix.shape[1]
    mesh = plsc.VectorSubcoreMesh(core_axis_name="c", subcore_axis_name="s")
    k = pl.kernel(
        _sc_body,
        mesh=mesh,
        compiler_params=pltpu.CompilerParams(needs_layout_passes=False),
        out_type=jax.ShapeDtypeStruct((b, d), relation_matrix.dtype),
        scratch_types=[
            pltpu.VMEM((b,), jnp.int32),
            pltpu.VMEM((_BPW,), jnp.int32),
            pltpu.VMEM((_BPW,), jnp.int32),
            pltpu.VMEM((_CHUNK, d), jnp.float32),
            pltpu.VMEM((_CHUNK, d), jnp.float32),
            pltpu.SemaphoreType.DMA,
            pltpu.SemaphoreType.DMA,
            pltpu.SemaphoreType.DMA,
            pltpu.SemaphoreType.DMA,
        ],
    )
    return k(relation_matrix, med, mole)


# speedup vs baseline: 5.4535x; 1.0009x over previous
import jax
import jax.numpy as jnp
from jax import lax
from jax.experimental import pallas as pl
from jax.experimental.pallas import tpu as pltpu
from jax.experimental.pallas import tpu_sc as plsc

_BPW = 128
_CHUNK = 32
_MAIN = 896
_TAIL = 104


def _sc_body(table_hbm, med_hbm, mole_hbm, out_hbm, med_v, mole_v, idx_v,
             bufA, bufB, tail_v, semA, semB, semt, semoA, semoB):
    nc = 2
    wid = lax.axis_index("s") * nc + lax.axis_index("c")
    base = wid * _BPW
    pltpu.sync_copy(med_hbm, med_v)
    pltpu.sync_copy(mole_hbm.at[pl.ds(base, _BPW)], mole_v)
    for t in range(_BPW // 16):
        m16 = mole_v[pl.ds(t * 16, 16)]
        idx_v[pl.ds(t * 16, 16)] = plsc.load_gather(med_v, [m16])

    tail_cps = []
    for t in range(_BPW // 16):
        vblk = idx_v[pl.ds(t * 16, 16)]
        for jj in range(16):
            g = vblk[jj]
            cp = pltpu.make_async_copy(
                table_hbm.at[g, pl.ds(_MAIN, _TAIL)],
                tail_v.at[t * 16 + jj], semt)
            cp.start()
            tail_cps.append(cp)

    bufs = (bufA, bufB)
    sems = (semA, semB)
    semos = (semoA, semoB)
    out_cps = [None, None]
    gathers = [None, None]
    n_chunks = _BPW // _CHUNK

    def fire(c):
        slot = c % 2
        cp = pltpu.make_async_copy(
            table_hbm.at[idx_v.at[pl.ds(c * _CHUNK, _CHUNK)],
                         pl.ds(0, _MAIN)],
            bufs[slot], sems[slot])
        cp.start()
        gathers[slot] = cp

    fire(0)
    for c in range(n_chunks):
        slot = c % 2
        gathers[slot].wait()
        if out_cps[slot] is not None:
            out_cps[slot].wait()
        oc = pltpu.make_async_copy(
            bufs[slot],
            out_hbm.at[pl.ds(base + c * _CHUNK, _CHUNK), pl.ds(0, _MAIN)],
            semos[slot])
        oc.start()
        out_cps[slot] = oc
        if c + 1 < n_chunks:
            nslot = (c + 1) % 2
            if out_cps[nslot] is not None:
                out_cps[nslot].wait()
                out_cps[nslot] = None
            fire(c + 1)
    for oc in out_cps:
        if oc is not None:
            oc.wait()

    for cp in tail_cps:
        cp.wait()
    pltpu.sync_copy(tail_v,
                    out_hbm.at[pl.ds(base, _BPW), pl.ds(_MAIN, _TAIL)])


def kernel(relation_matrix, med, mole):
    b = mole.shape[0]
    d = relation_matrix.shape[1]
    mesh = plsc.VectorSubcoreMesh(core_axis_name="c", subcore_axis_name="s")
    k = pl.kernel(
        _sc_body,
        mesh=mesh,
        compiler_params=pltpu.CompilerParams(needs_layout_passes=False),
        out_type=jax.ShapeDtypeStruct((b, d), relation_matrix.dtype),
        scratch_types=[
            pltpu.VMEM((b,), jnp.int32),
            pltpu.VMEM((_BPW,), jnp.int32),
            pltpu.VMEM((_BPW,), jnp.int32),
            pltpu.VMEM((_CHUNK, _MAIN), jnp.float32),
            pltpu.VMEM((_CHUNK, _MAIN), jnp.float32),
            pltpu.VMEM((_BPW, _TAIL), jnp.float32),
            pltpu.SemaphoreType.DMA,
            pltpu.SemaphoreType.DMA,
            pltpu.SemaphoreType.DMA,
            pltpu.SemaphoreType.DMA,
            pltpu.SemaphoreType.DMA,
        ],
    )
    return k(relation_matrix, med, mole)
